# baseline (speedup 1.0000x reference)
import jax
import jax.numpy as jnp
from jax import lax
from jax.experimental import pallas as pl
from jax.experimental.pallas import tpu as pltpu
from jax.experimental.pallas import tpu_sc as plsc

_NC = 2
_NS = 16
_NW = _NC * _NS
_L = 16

_B_BLK = 2048


def _make_sc_call(n_pos, widths, interpret=False):
    w0, w1, w2, w3 = widths
    d_out = w0 + w1 + w2 + w3
    assert n_pos % (_NW * _B_BLK) == 0
    p_per_w = n_pos // _NW
    n_blk = p_per_w // _B_BLK
    grp_per_blk = _B_BLK // _L
    x_rows_blk = _B_BLK * 4 // 128
    o_rows_blk = _B_BLK * d_out // 128
    assert _B_BLK * d_out % 128 == 0

    def body(x_hbm, t0_hbm, t1_hbm, t2_hbm, t3_hbm, out_hbm,
             t0v, t1v, t2v, t3v, xv, ov):
        wid = lax.axis_index("s") * _NC + lax.axis_index("c")
        base = wid * p_per_w
        pltpu.sync_copy(t0_hbm, t0v)
        pltpu.sync_copy(t1_hbm, t1v)
        pltpu.sync_copy(t2_hbm, t2v)
        pltpu.sync_copy(t3_hbm, t3v)

        def blk(b, carry):
            start = base + b * _B_BLK
            xr0 = pl.multiple_of(start * 4 // 128, 8)
            pltpu.sync_copy(x_hbm.at[pl.ds(xr0, x_rows_blk)], xv)

            def grp(g, c2):
                iota = lax.iota(jnp.int32, _L)
                xrow = jnp.broadcast_to(g // 2, (_L,))
                xcol = (g % 2) * 64 + iota * 4
                x0 = plsc.load_gather(xv, [xrow, xcol])
                x1 = plsc.load_gather(xv, [xrow, xcol + 1])
                x2 = plsc.load_gather(xv, [xrow, xcol + 2])
                x3 = plsc.load_gather(xv, [xrow, xcol + 3])
                ob = g * (d_out * _L) + iota * d_out
                a0 = x0 * w0
                a1 = x1 * w1
                a2 = x2 * w2
                a3 = x3 * w3
                addrs = ([a0 + j for j in range(w0)]
                         + [a1 + j for j in range(w1)]
                         + [a2 + j for j in range(w2)]
                         + [a3 + j for j in range(w3)])
                tabs = [t0v] * w0 + [t1v] * w1 + [t2v] * w2 + [t3v] * w3
                for j in range(d_out):
                    o = ob + j
                    plsc.store_scatter(ov, [o >> 7, o & 127],
                                       plsc.load_gather(tabs[j], [addrs[j]]))
                return c2

            lax.fori_loop(0, grp_per_blk, grp, 0)
            or0 = pl.multiple_of(start * d_out // 128, 8)
            pltpu.sync_copy(ov, out_hbm.at[pl.ds(or0, o_rows_blk)])
            return carry

        lax.fori_loop(0, n_blk, blk, 0)

    mesh = plsc.VectorSubcoreMesh(core_axis_name="c", subcore_axis_name="s",
                                  num_cores=_NC, num_subcores=_NS)
    return pl.kernel(
        body,
        out_type=jax.ShapeDtypeStruct((n_pos * d_out // 128, 128),
                                      jnp.float32),
        mesh=mesh,
        scratch_types=[
            pltpu.VMEM((11 * w0,), jnp.float32),
            pltpu.VMEM((18 * w1,), jnp.float32),
            pltpu.VMEM((24 * w2,), jnp.float32),
            pltpu.VMEM((7 * w3,), jnp.float32),
            pltpu.VMEM((x_rows_blk, 128), jnp.int32),
            pltpu.VMEM((o_rows_blk, 128), jnp.float32),
        ],
        compiler_params=pltpu.CompilerParams(needs_layout_passes=False),
        interpret=interpret,
    )


def kernel(x, W_wdir, W_weather, W_day, W_hour):
    b, t, _ = x.shape
    n_pos = b * t
    widths = (W_wdir.shape[1], W_weather.shape[1],
              W_day.shape[1], W_hour.shape[1])
    d_out = sum(widths)
    call = _make_sc_call(n_pos, widths)
    out = call(x.reshape(n_pos * 4 // 128, 128).astype(jnp.int32),
               W_wdir.reshape(-1), W_weather.reshape(-1),
               W_day.reshape(-1), W_hour.reshape(-1))
    return out.reshape(b, t, d_out)


# speedup vs baseline: 11.7170x; 11.7170x over previous
import jax
import jax.numpy as jnp
from jax import lax
from jax.experimental import pallas as pl
from jax.experimental.pallas import tpu as pltpu
from jax.experimental.pallas import tpu_sc as plsc

_NC = 2
_NS = 16
_NW = _NC * _NS
_L = 16


def _make_sc_call(n_b, n_t, widths):
    w = widths
    d_out = sum(w)
    nbt = n_b // 128
    ntt = n_t // 8
    tb_per_w = nbt // _NW
    x_rows = n_t * nbt * 4
    o_rows = d_out * ntt * nbt * 8
    o_rows_c = ntt * nbt * 8

    def body(x_hbm, t0_hbm, t1_hbm, t2_hbm, t3_hbm, out_hbm,
             t0v, t1v, t2v, t3v, xv, ov):
        wid = lax.axis_index("s") * _NC + lax.axis_index("c")
        tb0 = wid * tb_per_w
        pltpu.sync_copy(t0_hbm, t0v)
        pltpu.sync_copy(t1_hbm, t1v)
        pltpu.sync_copy(t2_hbm, t2v)
        pltpu.sync_copy(t3_hbm, t3v)
        tabs = (t0v, t1v, t2v, t3v)

        def blk(tt, carry):
            def in_dma(tr, c2):
                t = tt * 8 + tr
                r0 = pl.multiple_of(t * (4 * nbt) + tb0 * 4, 8)
                pltpu.sync_copy(x_hbm.at[pl.ds(r0, tb_per_w * 4)],
                                xv.at[tr])
                return c2
            lax.fori_loop(0, 8, in_dma, 0)

            def tr_loop(tr, c2):
                def tb_loop(tbl, c3):
                    orow = tbl * 8 + tr

                    def s_loop(s, c4):
                        col = s * _L
                        x0 = xv[tr, tbl * 4 + 0, pl.ds(col, _L)]
                        x1 = xv[tr, tbl * 4 + 1, pl.ds(col, _L)]
                        x2 = xv[tr, tbl * 4 + 2, pl.ds(col, _L)]
                        x3 = xv[tr, tbl * 4 + 3, pl.ds(col, _L)]
                        a = (x0 * w[0], x1 * w[1], x2 * w[2], x3 * w[3])
                        oc = 0
                        for ti in range(4):
                            for j in range(w[ti]):
                                v = plsc.load_gather(tabs[ti], [a[ti] + j])
                                ov[oc, orow, pl.ds(col, _L)] = v
                                oc += 1
                        return c4
                    lax.fori_loop(0, 128 // _L, s_loop, 0)
                    return c3
                lax.fori_loop(0, tb_per_w, tb_loop, 0)
                return c2
            lax.fori_loop(0, 8, tr_loop, 0)

            def out_dma(c15, c2):
                r0 = pl.multiple_of(
                    c15 * o_rows_c + tt * (nbt * 8) + tb0 * 8, 8)
                pltpu.sync_copy(ov.at[c15],
                                out_hbm.at[pl.ds(r0, tb_per_w * 8)])
                return c2
            lax.fori_loop(0, d_out, out_dma, 0)
            return carry

        lax.fori_loop(0, ntt, blk, 0)

    mesh = plsc.VectorSubcoreMesh(core_axis_name="c", subcore_axis_name="s",
                                  num_cores=_NC, num_subcores=_NS)
    return pl.kernel(
        body,
        out_type=jax.ShapeDtypeStruct((o_rows, 128), jnp.float32),
        mesh=mesh,
        scratch_types=[
            pltpu.VMEM((11 * w[0],), jnp.float32),
            pltpu.VMEM((18 * w[1],), jnp.float32),
            pltpu.VMEM((24 * w[2],), jnp.float32),
            pltpu.VMEM((7 * w[3],), jnp.float32),
            pltpu.VMEM((8, tb_per_w * 4, 128), jnp.int32),
            pltpu.VMEM((d_out, tb_per_w * 8, 128), jnp.float32),
        ],
        compiler_params=pltpu.CompilerParams(needs_layout_passes=False),
    )


def kernel(x, W_wdir, W_weather, W_day, W_hour):
    n_b, n_t, _ = x.shape
    widths = (W_wdir.shape[1], W_weather.shape[1],
              W_day.shape[1], W_hour.shape[1])
    d_out = sum(widths)
    nbt = n_b // 128
    ntt = n_t // 8

    xs = x.reshape(nbt, 128, n_t, 4)
    xp = xs.transpose(2, 0, 3, 1).reshape(n_t * nbt * 4, 128)

    call = _make_sc_call(n_b, n_t, widths)
    out = call(xp.astype(jnp.int32),
               W_wdir.reshape(-1), W_weather.reshape(-1),
               W_day.reshape(-1), W_hour.reshape(-1))

    o5 = out.reshape(d_out, ntt, nbt, 8, 128)
    return o5.transpose(2, 4, 1, 3, 0).reshape(n_b, n_t, d_out)


# speedup vs baseline: 16.8795x; 1.4406x over previous
import jax
import jax.numpy as jnp
from jax import lax
from jax.experimental import pallas as pl
from jax.experimental.pallas import tpu as pltpu
from jax.experimental.pallas import tpu_sc as plsc

_NC = 2
_NS = 16
_NW = _NC * _NS
_L = 16


def _make_sc_call(n_b, n_t, widths):
    w = widths
    d_out = sum(w)
    nbt = n_b // 128
    ntt = n_t // 8
    tb_per_w = nbt // _NW
    tb_blk = 2
    n_tbh = tb_per_w // tb_blk
    x_rows = n_t * nbt * 4
    o_rows = d_out * ntt * nbt * 8
    o_rows_c = ntt * nbt * 8

    def body(x_hbm, t0_hbm, t1_hbm, t2_hbm, t3_hbm, out_hbm,
             t0v, t1v, t2v, t3v, xv, ov, sin0, sin1, sout0, sout1):
        wid = lax.axis_index("s") * _NC + lax.axis_index("c")
        tb0 = wid * tb_per_w
        pltpu.sync_copy(t0_hbm, t0v)
        pltpu.sync_copy(t1_hbm, t1v)
        pltpu.sync_copy(t2_hbm, t2v)
        pltpu.sync_copy(t3_hbm, t3v)
        tabs = (t0v, t1v, t2v, t3v)
        sins = (sin0, sin1)
        souts = (sout0, sout1)

        def in_copies(tt, tbh, buf):
            for tr in range(8):
                r0 = pl.multiple_of(
                    (tt * 8 + tr) * (4 * nbt) + (tb0 + tbh * tb_blk) * 4, 8)
                yield pltpu.make_async_copy(
                    x_hbm.at[pl.ds(r0, tb_blk * 4)], xv.at[buf, tr],
                    sins[buf])

        def out_copies(tt, tbh, buf):
            for c15 in range(d_out):
                r0 = pl.multiple_of(
                    c15 * o_rows_c + tt * (nbt * 8)
                    + (tb0 + tbh * tb_blk) * 8, 8)
                yield pltpu.make_async_copy(
                    ov.at[buf, c15], out_hbm.at[pl.ds(r0, tb_blk * 8)],
                    souts[buf])

        def compute(tbh, buf):
            del tbh

            def tr_loop(tr, c2):
                def tb_loop(tbl, c3):
                    orow = tbl * 8 + tr

                    def s_loop(s, c4):
                        col = s * _L
                        x0 = xv[buf, tr, tbl * 4 + 0, pl.ds(col, _L)]
                        x1 = xv[buf, tr, tbl * 4 + 1, pl.ds(col, _L)]
                        x2 = xv[buf, tr, tbl * 4 + 2, pl.ds(col, _L)]
                        x3 = xv[buf, tr, tbl * 4 + 3, pl.ds(col, _L)]
                        a = (x0 * w[0], x1 * w[1], x2 * w[2], x3 * w[3])
                        oc = 0
                        for ti in range(4):
                            for j in range(w[ti]):
                                v = plsc.load_gather(tabs[ti], [a[ti] + j])
                                ov[buf, oc, orow, pl.ds(col, _L)] = v
                                oc += 1
                        return c4
                    lax.fori_loop(0, 128 // _L, s_loop, 0)
                    return c3
                lax.fori_loop(0, tb_blk, tb_loop, 0)
                return c2
            lax.fori_loop(0, 8, tr_loop, 0)

        for cp in in_copies(0, 0, 0):
            cp.start()

        def it_loop(it, carry):
            for phase in range(n_tbh):
                if phase + 1 < n_tbh:
                    for cp in in_copies(it, phase + 1, phase + 1):
                        cp.start()
                else:
                    @pl.when(it + 1 < ntt)
                    def _():
                        for cp in in_copies(it + 1, 0, 0):
                            cp.start()
                for cp in in_copies(it, phase, phase):
                    cp.wait()
                @pl.when(it > 0)
                def _():
                    for cp in out_copies(it - 1, phase, phase):
                        cp.wait()
                compute(phase, phase)
                for cp in out_copies(it, phase, phase):
                    cp.start()
            return carry

        lax.fori_loop(0, ntt, it_loop, 0)

        for phase in range(n_tbh):
            for cp in out_copies(ntt - 1, phase, phase):
                cp.wait()

    mesh = plsc.VectorSubcoreMesh(core_axis_name="c", subcore_axis_name="s",
                                  num_cores=_NC, num_subcores=_NS)
    return pl.kernel(
        body,
        out_type=jax.ShapeDtypeStruct((o_rows, 128), jnp.float32),
        mesh=mesh,
        scratch_types=[
            pltpu.VMEM((11 * w[0],), jnp.float32),
            pltpu.VMEM((18 * w[1],), jnp.float32),
            pltpu.VMEM((24 * w[2],), jnp.float32),
            pltpu.VMEM((7 * w[3],), jnp.float32),
            pltpu.VMEM((2, 8, tb_blk * 4, 128), jnp.int32),
            pltpu.VMEM((2, d_out, tb_blk * 8, 128), jnp.float32),
            pltpu.SemaphoreType.DMA,
            pltpu.SemaphoreType.DMA,
            pltpu.SemaphoreType.DMA,
            pltpu.SemaphoreType.DMA,
        ],
        compiler_params=pltpu.CompilerParams(needs_layout_passes=False),
    )


def kernel(x, W_wdir, W_weather, W_day, W_hour):
    n_b, n_t, _ = x.shape
    widths = (W_wdir.shape[1], W_weather.shape[1],
              W_day.shape[1], W_hour.shape[1])
    d_out = sum(widths)
    nbt = n_b // 128
    ntt = n_t // 8

    xs = x.reshape(nbt, 128, n_t, 4)
    xp = xs.transpose(2, 0, 3, 1).reshape(n_t * nbt * 4, 128)

    call = _make_sc_call(n_b, n_t, widths)
    out = call(xp.astype(jnp.int32),
               W_wdir.reshape(-1), W_weather.reshape(-1),
               W_day.reshape(-1), W_hour.reshape(-1))

    o5 = out.reshape(d_out, ntt, nbt, 8, 128)
    return o5.transpose(2, 4, 1, 3, 0).reshape(n_b, n_t, d_out)


# speedup vs baseline: 39.4875x; 2.3394x over previous
import jax
import jax.numpy as jnp
from jax import lax
from jax.experimental import pallas as pl
from jax.experimental.pallas import tpu as pltpu
from jax.experimental.pallas import tpu_sc as plsc

_NC = 2
_NS = 16
_NW = _NC * _NS
_L = 16


def _make_sc_call(n_b, n_t, widths):
    w = widths
    d_out = sum(w)
    nbt = n_b // 128
    ntt = n_t // 8
    tb_per_w = nbt // _NW
    tb_blk = 2
    n_tbh = tb_per_w // tb_blk
    x_rows = n_t * nbt * 4
    o_rows = d_out * ntt * nbt * 8
    o_rows_c = ntt * nbt * 8

    def body(x_hbm, t0_hbm, t1_hbm, t2_hbm, t3_hbm, out_hbm,
             t0v, t1v, t2v, t3v, xv, ov, sin0, sin1, sout0, sout1):
        wid = lax.axis_index("s") * _NC + lax.axis_index("c")
        tb0 = wid * tb_per_w
        pltpu.sync_copy(t0_hbm, t0v)
        pltpu.sync_copy(t1_hbm, t1v)
        pltpu.sync_copy(t2_hbm, t2v)
        pltpu.sync_copy(t3_hbm, t3v)
        tabs = (t0v, t1v, t2v, t3v)
        sins = (sin0, sin1)
        souts = (sout0, sout1)

        def in_copies(tt, tbh, buf):
            for tr in range(8):
                r0 = pl.multiple_of(
                    (tt * 8 + tr) * (4 * nbt) + (tb0 + tbh * tb_blk) * 4, 8)
                yield pltpu.make_async_copy(
                    x_hbm.at[pl.ds(r0, tb_blk * 4)], xv.at[buf, tr],
                    sins[buf])

        def out_copies(tt, tbh, buf):
            for c15 in range(d_out):
                r0 = pl.multiple_of(
                    c15 * o_rows_c + tt * (nbt * 8)
                    + (tb0 + tbh * tb_blk) * 8, 8)
                yield pltpu.make_async_copy(
                    ov.at[buf, c15], out_hbm.at[pl.ds(r0, tb_blk * 8)],
                    souts[buf])

        def compute(tbh, buf):
            del tbh

            def tr_loop(tr, c2):
                def tb_loop(tbl, c3):
                    orow = tbl * 8 + tr

                    def s_loop(s, c4):
                        col = s * _L
                        x0 = xv[buf, tr, tbl * 4 + 0, pl.ds(col, _L)]
                        x1 = xv[buf, tr, tbl * 4 + 1, pl.ds(col, _L)]
                        x2 = xv[buf, tr, tbl * 4 + 2, pl.ds(col, _L)]
                        x3 = xv[buf, tr, tbl * 4 + 3, pl.ds(col, _L)]
                        a = (x0 * w[0], x1 * w[1], x2 * w[2], x3 * w[3])
                        vals = []
                        for ti in range(4):
                            for j in range(w[ti]):
                                vals.append(
                                    plsc.load_gather(tabs[ti], [a[ti] + j]))
                        for oc, v in enumerate(vals):
                            ov[buf, oc, orow, pl.ds(col, _L)] = v
                        return c4
                    lax.fori_loop(0, 128 // _L, s_loop, 0)
                    return c3
                lax.fori_loop(0, tb_blk, tb_loop, 0)
                return c2
            lax.fori_loop(0, 8, tr_loop, 0)

        for cp in in_copies(0, 0, 0):
            cp.start()

        def it_loop(it, carry):
            for phase in range(n_tbh):
                if phase + 1 < n_tbh:
                    for cp in in_copies(it, phase + 1, phase + 1):
                        cp.start()
                else:
                    @pl.when(it + 1 < ntt)
                    def _():
                        for cp in in_copies(it + 1, 0, 0):
                            cp.start()
                for cp in in_copies(it, phase, phase):
                    cp.wait()
                @pl.when(it > 0)
                def _():
                    for cp in out_copies(it - 1, phase, phase):
                        cp.wait()
                compute(phase, phase)
                for cp in out_copies(it, phase, phase):
                    cp.start()
            return carry

        lax.fori_loop(0, ntt, it_loop, 0)

        for phase in range(n_tbh):
            for cp in out_copies(ntt - 1, phase, phase):
                cp.wait()

    mesh = plsc.VectorSubcoreMesh(core_axis_name="c", subcore_axis_name="s",
                                  num_cores=_NC, num_subcores=_NS)
    return pl.kernel(
        body,
        out_type=jax.ShapeDtypeStruct((o_rows, 128), jnp.float32),
        mesh=mesh,
        scratch_types=[
            pltpu.VMEM((11 * w[0],), jnp.float32),
            pltpu.VMEM((18 * w[1],), jnp.float32),
            pltpu.VMEM((24 * w[2],), jnp.float32),
            pltpu.VMEM((7 * w[3],), jnp.float32),
            pltpu.VMEM((2, 8, tb_blk * 4, 128), jnp.int32),
            pltpu.VMEM((2, d_out, tb_blk * 8, 128), jnp.float32),
            pltpu.SemaphoreType.DMA,
            pltpu.SemaphoreType.DMA,
            pltpu.SemaphoreType.DMA,
            pltpu.SemaphoreType.DMA,
        ],
        compiler_params=pltpu.CompilerParams(needs_layout_passes=False),
    )


def kernel(x, W_wdir, W_weather, W_day, W_hour):
    n_b, n_t, _ = x.shape
    widths = (W_wdir.shape[1], W_weather.shape[1],
              W_day.shape[1], W_hour.shape[1])
    d_out = sum(widths)
    nbt = n_b // 128
    ntt = n_t // 8

    xs = x.reshape(nbt, 128, n_t, 4)
    xp = xs.transpose(2, 0, 3, 1).reshape(n_t * nbt * 4, 128)

    call = _make_sc_call(n_b, n_t, widths)
    out = call(xp.astype(jnp.int32),
               W_wdir.reshape(-1), W_weather.reshape(-1),
               W_day.reshape(-1), W_hour.reshape(-1))

    o5 = out.reshape(d_out, ntt, nbt, 8, 128)
    return o5.transpose(2, 4, 1, 3, 0).reshape(n_b, n_t, d_out)


# speedup vs baseline: 62.5372x; 1.5837x over previous
import jax
import jax.numpy as jnp
from jax import lax
from jax.experimental import pallas as pl
from jax.experimental.pallas import tpu as pltpu
from jax.experimental.pallas import tpu_sc as plsc

_NC = 2
_NS = 16
_NW = _NC * _NS
_L = 16


def _make_sc_call(n_b, n_t, widths):
    w = widths
    d_out = sum(w)
    nbt = n_b // 128
    ntt = n_t // 8
    tb_per_w = nbt // _NW
    tb_blk = 2
    n_tbh = tb_per_w // tb_blk
    x_rows = n_t * nbt * 4
    o_rows = d_out * ntt * nbt * 8
    o_rows_c = ntt * nbt * 8

    def body(x_hbm, t0_hbm, t1_hbm, t2_hbm, t3_hbm, out_hbm,
             t0v, t1v, t2v, t3v, xv, ov, sin0, sin1, sout0, sout1):
        wid = lax.axis_index("s") * _NC + lax.axis_index("c")
        tb0 = wid * tb_per_w
        pltpu.sync_copy(t0_hbm, t0v)
        pltpu.sync_copy(t1_hbm, t1v)
        pltpu.sync_copy(t2_hbm, t2v)
        pltpu.sync_copy(t3_hbm, t3v)
        tabs = (t0v, t1v, t2v, t3v)
        sins = (sin0, sin1)
        souts = (sout0, sout1)

        def in_copies(tt, tbh, buf):
            for tr in range(8):
                r0 = pl.multiple_of(
                    (tt * 8 + tr) * (4 * nbt) + (tb0 + tbh * tb_blk) * 4, 8)
                yield pltpu.make_async_copy(
                    x_hbm.at[pl.ds(r0, tb_blk * 4)], xv.at[buf, tr],
                    sins[buf])

        def out_copies(tt, tbh, buf):
            for c15 in range(d_out):
                r0 = pl.multiple_of(
                    c15 * o_rows_c + tt * (nbt * 8)
                    + (tb0 + tbh * tb_blk) * 8, 8)
                yield pltpu.make_async_copy(
                    ov.at[buf, c15], out_hbm.at[pl.ds(r0, tb_blk * 8)],
                    souts[buf])

        def compute(buf):
            @plsc.parallel_loop(0, 8 * tb_blk * (128 // _L), unroll=2)
            def _sub(i):
                s = i % (128 // _L)
                tbl = (i // (128 // _L)) % tb_blk
                tr = i // ((128 // _L) * tb_blk)
                orow = tbl * 8 + tr
                col = s * _L
                xc = [xv[buf, tr, tbl * 4 + ti, pl.ds(col, _L)]
                      for ti in range(4)]
                a = [xc[ti] * w[ti] for ti in range(4)]
                vals = []
                for ti in range(4):
                    for j in range(w[ti]):
                        vals.append(
                            plsc.load_gather(tabs[ti], [a[ti] + j]))
                for oc, v in enumerate(vals):
                    ov[buf, oc, orow, pl.ds(col, _L)] = v

        for cp in in_copies(0, 0, 0):
            cp.start()

        def it_loop(it, carry):
            for phase in range(n_tbh):
                if phase + 1 < n_tbh:
                    for cp in in_copies(it, phase + 1, phase + 1):
                        cp.start()
                else:
                    @pl.when(it + 1 < ntt)
                    def _():
                        for cp in in_copies(it + 1, 0, 0):
                            cp.start()
                for cp in in_copies(it, phase, phase):
                    cp.wait()
                @pl.when(it > 0)
                def _():
                    for cp in out_copies(it - 1, phase, phase):
                        cp.wait()
                compute(phase)
                for cp in out_copies(it, phase, phase):
                    cp.start()
            return carry

        lax.fori_loop(0, ntt, it_loop, 0)

        for phase in range(n_tbh):
            for cp in out_copies(ntt - 1, phase, phase):
                cp.wait()

    mesh = plsc.VectorSubcoreMesh(core_axis_name="c", subcore_axis_name="s",
                                  num_cores=_NC, num_subcores=_NS)
    return pl.kernel(
        body,
        out_type=jax.ShapeDtypeStruct((o_rows, 128), jnp.float32),
        mesh=mesh,
        scratch_types=[
            pltpu.VMEM((11 * w[0],), jnp.float32),
            pltpu.VMEM((18 * w[1],), jnp.float32),
            pltpu.VMEM((24 * w[2],), jnp.float32),
            pltpu.VMEM((7 * w[3],), jnp.float32),
            pltpu.VMEM((2, 8, tb_blk * 4, 128), jnp.int32),
            pltpu.VMEM((2, d_out, tb_blk * 8, 128), jnp.float32),
            pltpu.SemaphoreType.DMA,
            pltpu.SemaphoreType.DMA,
            pltpu.SemaphoreType.DMA,
            pltpu.SemaphoreType.DMA,
        ],
        compiler_params=pltpu.CompilerParams(needs_layout_passes=False),
    )


def kernel(x, W_wdir, W_weather, W_day, W_hour):
    n_b, n_t, _ = x.shape
    widths = (W_wdir.shape[1], W_weather.shape[1],
              W_day.shape[1], W_hour.shape[1])
    d_out = sum(widths)
    nbt = n_b // 128
    ntt = n_t // 8

    xs = x.reshape(nbt, 128, n_t, 4)
    xp = xs.transpose(2, 0, 3, 1).reshape(n_t * nbt * 4, 128)

    call = _make_sc_call(n_b, n_t, widths)
    out = call(xp.astype(jnp.int32),
               W_wdir.reshape(-1), W_weather.reshape(-1),
               W_day.reshape(-1), W_hour.reshape(-1))

    o5 = out.reshape(d_out, ntt, nbt, 8, 128)
    return o5.transpose(2, 4, 1, 3, 0).reshape(n_b, n_t, d_out)
